# Initial kernel scaffold; baseline (speedup 1.0000x reference)
import jax
import jax.numpy as jnp
from jax.experimental import pallas as pl


def kernel(x, edge_index, batch, W_in, b_in, Wl0, Wr0, att0, bg0, lng0, lnb0, Wl1, Wr1, att1, bg1, lng1, lnb1, W1, b1, W2, b2, W3, b3):
    raise NotImplementedError("write your pallas kernel here")


# speedup vs baseline: 28.6683x; 28.6683x over previous
import functools

import jax
import jax.numpy as jnp
from jax import lax
from jax.experimental import pallas as pl
from jax.experimental.pallas import tpu as pltpu
from jax.experimental.pallas import tpu_sc as plsc

N = 50000
E = 800000
F_IN = 2
HID = 32
H = 2
C = 32
NG = 64

N8 = 50176
ROWB = 1024
NRB = N8 // ROWB
STRIPE = N8 // 16
ZROWS = STRIPE // 28
DENW = 16

CHUNK = 256
NSUB = 16
CPS = -(-(E + N) // (NSUB * CHUNK))
EP = NSUB * CPS * CHUNK
AUGW = 36
EB = 2048
NEB = EP // EB
CHUNK_S = 128
CPS_S = EP // (NSUB * CHUNK_S)

_f32 = jnp.float32


def _proj_write(h, wl_ref, wr_ref, xlp_ref, xrp_ref):
    xl = jnp.dot(h, wl_ref[...], preferred_element_type=_f32)
    xr = jnp.dot(h, wr_ref[...], preferred_element_type=_f32)
    xlp_ref[...] = xl.reshape(ROWB, H, C).transpose(1, 0, 2)
    xrp_ref[...] = xr.reshape(ROWB, H, C).transpose(1, 0, 2)


def _dense_pre_body(x_ref, win_ref, bin_ref, wl_ref, wr_ref,
                    xlp_ref, xrp_ref):
    i = pl.program_id(0)
    rows = i * ROWB + lax.broadcasted_iota(jnp.int32, (ROWB, 1), 0)
    valid = rows < N
    h = jax.nn.relu(jnp.dot(x_ref[...], win_ref[...],
                            preferred_element_type=_f32) + bin_ref[...])
    h = jnp.where(valid, h, 0.0)
    _proj_write(h, wl_ref, wr_ref, xlp_ref, xrp_ref)


def _node_update(parts, den3, bg_ref, lng_ref, lnb_ref, valid):
    num = parts
    den = den3[:, :, 0:1]
    o = num / (den + 1e-16)
    t = (o[0] + o[1]) * 0.5 + bg_ref[...]
    mu = jnp.mean(t, axis=-1, keepdims=True)
    var = jnp.mean(jnp.square(t - mu), axis=-1, keepdims=True)
    hn = (t - mu) / jnp.sqrt(var + 1e-5) * lng_ref[...] + lnb_ref[...]
    hn = jax.nn.relu(hn)
    return jnp.where(valid, hn, 0.0)


def _dense_mid_body(parts_ref, den_ref, bg_ref, lng_ref, lnb_ref,
                    wl_ref, wr_ref, xlp_ref, xrp_ref):
    i = pl.program_id(0)
    rows = i * ROWB + lax.broadcasted_iota(jnp.int32, (ROWB, 1), 0)
    valid = rows < N
    h = _node_update(parts_ref[...], den_ref[...], bg_ref, lng_ref, lnb_ref,
                     valid)
    _proj_write(h, wl_ref, wr_ref, xlp_ref, xrp_ref)


def _dense_post_body(parts_ref, den_ref, bg_ref, lng_ref, lnb_ref, h_ref):
    i = pl.program_id(0)
    rows = i * ROWB + lax.broadcasted_iota(jnp.int32, (ROWB, 1), 0)
    valid = rows < N
    h_ref[...] = _node_update(parts_ref[...], den_ref[...], bg_ref, lng_ref,
                              lnb_ref, valid)


def _logit_body(gsum_ref, att_ref, a_ref):
    g = gsum_ref[...]
    att = att_ref[...]
    l0 = jnp.sum(g[0] * att[0][None, :], axis=-1)
    l1 = jnp.sum(g[1] * att[1][None, :], axis=-1)
    lg = jnp.stack([l0, l1])
    a_ref[...] = jnp.exp(jnp.minimum(lg, 50.0))


def _pool_mlp_body(h_ref, b3d_ref, w1_ref, b1_ref, w2_ref, b2_ref,
                   w3_ref, b3_ref, out_ref, acc_ref):
    i = pl.program_id(0)

    @pl.when(i == 0)
    def _():
        acc_ref[...] = jnp.zeros_like(acc_ref)

    b = b3d_ref[0, 0, :]
    onehot = (lax.broadcasted_iota(jnp.int32, (NG, ROWB), 0)
              == b[None, :]).astype(_f32)
    haug = jnp.concatenate(
        [h_ref[...], jnp.ones((ROWB, 1), _f32)], axis=-1)
    acc_ref[...] += jnp.dot(onehot, haug, preferred_element_type=_f32)

    @pl.when(i == NRB - 1)
    def _():
        acc = acc_ref[...]
        emb = acc[:, :HID] / jnp.maximum(acc[:, HID:HID + 1], 1.0)
        z = jax.nn.relu(jnp.dot(emb, w1_ref[...],
                                preferred_element_type=_f32) + b1_ref[...])
        z = jax.nn.relu(jnp.dot(z, w2_ref[...],
                                preferred_element_type=_f32) + b2_ref[...])
        z = jnp.dot(z, w3_ref[...], preferred_element_type=_f32) + b3_ref[...]
        out_ref[...] = jax.nn.sigmoid(z)


def _sc_gather_body(xl_hbm, xr_hbm, s_hbm, d_hbm, gsum_hbm,
                    sidx_v, didx_v, bufl, bufr, seml, semr):
    c = lax.axis_index("c")
    s = lax.axis_index("s")
    off = c * N8

    def chunk_body(j, carry):
        base = (s * CPS + j) * CHUNK
        pltpu.sync_copy(s_hbm.at[pl.ds(base, CHUNK)], sidx_v)
        pltpu.sync_copy(d_hbm.at[pl.ds(base, CHUNK)], didx_v)

        def adj(t, carry2):
            sl = pl.ds(t * 16, 16)
            sidx_v[sl] = sidx_v[sl] + off
            didx_v[sl] = didx_v[sl] + off
            return carry2
        lax.fori_loop(0, CHUNK // 16, adj, 0)

        cpl = pltpu.async_copy(xl_hbm.at[sidx_v], bufl, seml)
        cpr = pltpu.async_copy(xr_hbm.at[didx_v], bufr, semr)
        cpl.wait()
        cpr.wait()

        def row(k, carry2):
            for o in (0, 16):
                sl = pl.ds(o, 16)
                v = bufl[k, sl] + bufr[k, sl]
                bufl[k, sl] = jnp.maximum(v, v * 0.2)
            return carry2
        lax.fori_loop(0, CHUNK, row, 0)

        pltpu.sync_copy(bufl, gsum_hbm.at[pl.ds(c * EP + base, CHUNK)])
        return carry
    lax.fori_loop(0, CPS, chunk_body, 0)


def _sc_scatter_body(xl_hbm, s_hbm, d_hbm, a_hbm, parts_hbm,
                     sidx_v, didx_v, a_v, gbuf, zbuf, acc, sem):
    c = lax.axis_index("c")
    s = lax.axis_index("s")
    off = c * N8

    def zrow(i, carry):
        z = jnp.zeros((16,), _f32)
        zbuf[i, pl.ds(0, 16)] = z
        zbuf[i, pl.ds(16, 16)] = z
        return carry
    lax.fori_loop(0, ZROWS, zrow, 0)
    for r in range(28):
        pltpu.sync_copy(zbuf, acc.at[pl.ds(s * STRIPE + r * ZROWS, ZROWS)])
    plsc.subcore_barrier()

    def chunk_body(j, carry):
        base = (s * CPS_S + j) * CHUNK_S
        pltpu.sync_copy(s_hbm.at[pl.ds(base, CHUNK_S)], sidx_v)
        pltpu.sync_copy(d_hbm.at[pl.ds(base, CHUNK_S)], didx_v)
        pltpu.sync_copy(a_hbm.at[pl.ds(c * EP + base, CHUNK_S)], a_v)

        def adj(t, carry2):
            sl = pl.ds(t * 16, 16)
            sidx_v[sl] = sidx_v[sl] + off
            return carry2
        lax.fori_loop(0, CHUNK_S // 16, adj, 0)

        pltpu.async_copy(xl_hbm.at[sidx_v], gbuf, sem).wait()

        def grp(t, carry2):
            av = a_v[pl.ds(t * 16, 16)]
            s0, s1 = pl.ds(0, 16), pl.ds(16, 16)
            for i in range(16):
                k = t * 16 + i
                ak = av[i]
                v0, v1 = gbuf[k, s0], gbuf[k, s1]
                gbuf[k, s0] = v0 * ak
                gbuf[k, s1] = v1 * ak
            return carry2
        lax.fori_loop(0, CHUNK_S // 16, grp, 0)

        pltpu.sync_copy(gbuf, acc.at[didx_v], add=True)
        return carry
    lax.fori_loop(0, CPS_S, chunk_body, 0)

    plsc.subcore_barrier()
    pltpu.sync_copy(acc.at[pl.ds(s * STRIPE, STRIPE)],
                    parts_hbm.at[pl.ds(c * N8 + s * STRIPE, STRIPE)])


def _sc_den_body(d_hbm, a_hbm, den_hbm, didx_v, a_v, denbuf, zbuf, acc):
    c = lax.axis_index("c")
    s = lax.axis_index("s")
    iota16 = jnp.arange(16, dtype=jnp.int32)
    zero16 = jnp.zeros((16,), jnp.int32)

    def zrow(i, carry):
        zbuf[i, pl.ds(0, 16)] = jnp.zeros((16,), _f32)
        return carry
    lax.fori_loop(0, ZROWS, zrow, 0)

    def zden(i, carry):
        denbuf[i, pl.ds(0, 16)] = jnp.zeros((16,), _f32)
        return carry
    lax.fori_loop(0, CHUNK_S, zden, 0)
    for r in range(28):
        pltpu.sync_copy(zbuf, acc.at[pl.ds(s * STRIPE + r * ZROWS, ZROWS)])
    plsc.subcore_barrier()

    def chunk_body(j, carry):
        base = (s * CPS_S + j) * CHUNK_S
        pltpu.sync_copy(d_hbm.at[pl.ds(base, CHUNK_S)], didx_v)
        pltpu.sync_copy(a_hbm.at[pl.ds(c * EP + base, CHUNK_S)], a_v)

        def grp(t, carry2):
            av = a_v[pl.ds(t * 16, 16)]
            plsc.store_scatter(denbuf, [iota16 + t * 16, zero16], av)
            return carry2
        lax.fori_loop(0, CHUNK_S // 16, grp, 0)

        pltpu.sync_copy(denbuf, acc.at[didx_v], add=True)
        return carry
    lax.fori_loop(0, CPS_S, chunk_body, 0)

    plsc.subcore_barrier()
    pltpu.sync_copy(acc.at[pl.ds(s * STRIPE, STRIPE)],
                    den_hbm.at[pl.ds(c * N8 + s * STRIPE, STRIPE)])


_SC_MESH = plsc.VectorSubcoreMesh(core_axis_name="c", subcore_axis_name="s")
_SC_PARAMS = pltpu.CompilerParams(use_tc_tiling_on_sc=False)

_sc_gather = pl.kernel(
    _sc_gather_body,
    out_type=jax.ShapeDtypeStruct((H * EP, C), _f32),
    mesh=_SC_MESH,
    compiler_params=_SC_PARAMS,
    scratch_types=[
        pltpu.VMEM((CHUNK,), jnp.int32),
        pltpu.VMEM((CHUNK,), jnp.int32),
        pltpu.VMEM((CHUNK, C), _f32),
        pltpu.VMEM((CHUNK, C), _f32),
        pltpu.SemaphoreType.DMA,
        pltpu.SemaphoreType.DMA,
    ],
)

_sc_scatter = pl.kernel(
    _sc_scatter_body,
    out_type=jax.ShapeDtypeStruct((H * N8, C), _f32),
    mesh=_SC_MESH,
    compiler_params=_SC_PARAMS,
    scratch_types=[
        pltpu.VMEM((CHUNK_S,), jnp.int32),
        pltpu.VMEM((CHUNK_S,), jnp.int32),
        pltpu.VMEM((CHUNK_S,), _f32),
        pltpu.VMEM((CHUNK_S, C), _f32),
        pltpu.VMEM((ZROWS, C), _f32),
        pltpu.VMEM_SHARED((N8, C), _f32),
        pltpu.SemaphoreType.DMA,
    ],
)

_sc_den = pl.kernel(
    _sc_den_body,
    out_type=jax.ShapeDtypeStruct((H * N8, DENW), _f32),
    mesh=_SC_MESH,
    compiler_params=pltpu.CompilerParams(
        use_tc_tiling_on_sc=False, needs_layout_passes=False),
    scratch_types=[
        pltpu.VMEM((CHUNK_S,), jnp.int32),
        pltpu.VMEM((CHUNK_S,), _f32),
        pltpu.VMEM((CHUNK_S, DENW), _f32),
        pltpu.VMEM((ZROWS, DENW), _f32),
        pltpu.VMEM_SHARED((N8, DENW), _f32),
    ],
)


def _full(shape):
    return pl.BlockSpec(shape, lambda i: tuple(0 for _ in shape))


def _dense_pre(x_pad, w_in, b_in, wl, wr):
    return pl.pallas_call(
        _dense_pre_body,
        grid=(NRB,),
        in_specs=[
            pl.BlockSpec((ROWB, F_IN), lambda i: (i, 0)),
            _full((F_IN, HID)),
            _full((1, HID)),
            _full((HID, H * C)),
            _full((HID, H * C)),
        ],
        out_specs=[
            pl.BlockSpec((H, ROWB, C), lambda i: (0, i, 0)),
            pl.BlockSpec((H, ROWB, C), lambda i: (0, i, 0)),
        ],
        out_shape=[
            jax.ShapeDtypeStruct((H, N8, C), _f32),
            jax.ShapeDtypeStruct((H, N8, C), _f32),
        ],
    )(x_pad, w_in, b_in, wl, wr)


def _dense_mid(parts, den, bg, lng, lnb, wl, wr):
    return pl.pallas_call(
        _dense_mid_body,
        grid=(NRB,),
        in_specs=[
            pl.BlockSpec((H, ROWB, C), lambda i: (0, i, 0)),
            pl.BlockSpec((H, ROWB, DENW), lambda i: (0, i, 0)),
            _full((1, C)),
            _full((1, HID)),
            _full((1, HID)),
            _full((HID, H * C)),
            _full((HID, H * C)),
        ],
        out_specs=[
            pl.BlockSpec((H, ROWB, C), lambda i: (0, i, 0)),
            pl.BlockSpec((H, ROWB, C), lambda i: (0, i, 0)),
        ],
        out_shape=[
            jax.ShapeDtypeStruct((H, N8, C), _f32),
            jax.ShapeDtypeStruct((H, N8, C), _f32),
        ],
    )(parts, den, bg, lng, lnb, wl, wr)


def _dense_post(parts, den, bg, lng, lnb):
    return pl.pallas_call(
        _dense_post_body,
        grid=(NRB,),
        in_specs=[
            pl.BlockSpec((H, ROWB, C), lambda i: (0, i, 0)),
            pl.BlockSpec((H, ROWB, DENW), lambda i: (0, i, 0)),
            _full((1, C)),
            _full((1, HID)),
            _full((1, HID)),
        ],
        out_specs=pl.BlockSpec((ROWB, HID), lambda i: (i, 0)),
        out_shape=jax.ShapeDtypeStruct((N8, HID), _f32),
    )(parts, den, bg, lng, lnb)


def _logits(gsum, att):
    return pl.pallas_call(
        _logit_body,
        grid=(NEB,),
        in_specs=[
            pl.BlockSpec((H, EB, C), lambda i: (0, i, 0)),
            _full((H, C)),
        ],
        out_specs=pl.BlockSpec((H, EB), lambda i: (0, i)),
        out_shape=jax.ShapeDtypeStruct((H, EP), _f32),
    )(gsum, att)


def _pool_mlp(h2, batch3d, w1, b1, w2, b2, w3, b3):
    return pl.pallas_call(
        _pool_mlp_body,
        grid=(NRB,),
        in_specs=[
            pl.BlockSpec((ROWB, HID), lambda i: (i, 0)),
            pl.BlockSpec((1, 1, ROWB), lambda i: (i, 0, 0)),
            _full((HID, 128)),
            _full((1, 128)),
            _full((128, 64)),
            _full((1, 64)),
            _full((64, 1)),
            _full((1, 1)),
        ],
        out_specs=pl.BlockSpec((NG, 1), lambda i: (0, 0)),
        out_shape=jax.ShapeDtypeStruct((NG, 1), _f32),
        scratch_shapes=[pltpu.VMEM((NG, HID + 1), _f32)],
    )(h2, batch3d, w1, b1, w2, b2, w3, b3)


def _gat_layer(xlp, xrp, sidx, didx, att):
    xl_flat = xlp.reshape(H * N8, C)
    xr_flat = xrp.reshape(H * N8, C)
    gsum = _sc_gather(xl_flat, xr_flat, sidx, didx)
    a = _logits(gsum.reshape(H, EP, C), att).reshape(H * EP)
    parts = _sc_scatter(xl_flat, sidx, didx, a)
    den = _sc_den(didx, a)
    return parts.reshape(H, N8, C), den.reshape(H, N8, DENW)


def kernel(x, edge_index, batch, W_in, b_in, Wl0, Wr0, att0, bg0, lng0, lnb0,
           Wl1, Wr1, att1, bg1, lng1, lnb1, W1, b1, W2, b2, W3, b3):
    loop = jnp.arange(N, dtype=jnp.int32)
    padv = jnp.full((EP - E - N,), N, jnp.int32)
    sidx = jnp.concatenate([edge_index[0], loop, padv])
    didx = jnp.concatenate([edge_index[1], loop, padv])
    x_pad = jnp.pad(x, ((0, N8 - N), (0, 0)))
    batch3d = jnp.concatenate(
        [batch, jnp.full((N8 - N,), NG, jnp.int32)]).reshape(NRB, 1, ROWB)

    xlp0, xrp0 = _dense_pre(x_pad, W_in, b_in.reshape(1, HID), Wl0, Wr0)
    parts0, den0 = _gat_layer(xlp0, xrp0, sidx, didx, att0)

    xlp1, xrp1 = _dense_mid(
        parts0, den0, bg0.reshape(1, C), lng0.reshape(1, HID),
        lnb0.reshape(1, HID), Wl1, Wr1)
    parts1, den1 = _gat_layer(xlp1, xrp1, sidx, didx, att1)

    h2 = _dense_post(
        parts1, den1, bg1.reshape(1, C), lng1.reshape(1, HID),
        lnb1.reshape(1, HID))

    return _pool_mlp(h2, batch3d, W1, b1.reshape(1, 128), W2,
                     b2.reshape(1, 64), W3, b3.reshape(1, 1))


# speedup vs baseline: 36.8782x; 1.2864x over previous
import functools

import jax
import jax.numpy as jnp
from jax import lax
from jax.experimental import pallas as pl
from jax.experimental.pallas import tpu as pltpu
from jax.experimental.pallas import tpu_sc as plsc

N = 50000
E = 800000
F_IN = 2
HID = 32
H = 2
C = 32
NG = 64

N8 = 50176
ROWB = 1024
NRB = N8 // ROWB
STRIPE = N8 // 16
ZROWS = STRIPE // 28
DENW = 16

CHUNK = 256
NSUB = 16
CPS = -(-(E + N) // (NSUB * CHUNK))
EP = NSUB * CPS * CHUNK
CHUNK_G = 512
CPS_G = EP // (NSUB * CHUNK_G)
AUGW = 36
EB = 2048
NEB = EP // EB
CHUNK_S = 128
CPS_S = EP // (NSUB * CHUNK_S)

_f32 = jnp.float32


def _proj_write(h, wl_ref, wr_ref, xlp_ref, xrp_ref):
    xl = jnp.dot(h, wl_ref[...], preferred_element_type=_f32)
    xr = jnp.dot(h, wr_ref[...], preferred_element_type=_f32)
    xlp_ref[...] = xl.reshape(ROWB, H, C).transpose(1, 0, 2)
    xrp_ref[...] = xr.reshape(ROWB, H, C).transpose(1, 0, 2)


def _dense_pre_body(x_ref, win_ref, bin_ref, wl_ref, wr_ref,
                    xlp_ref, xrp_ref):
    i = pl.program_id(0)
    rows = i * ROWB + lax.broadcasted_iota(jnp.int32, (ROWB, 1), 0)
    valid = rows < N
    h = jax.nn.relu(jnp.dot(x_ref[...], win_ref[...],
                            preferred_element_type=_f32) + bin_ref[...])
    h = jnp.where(valid, h, 0.0)
    _proj_write(h, wl_ref, wr_ref, xlp_ref, xrp_ref)


def _node_update(parts, den3, bg_ref, lng_ref, lnb_ref, valid):
    num = parts
    den = den3[:, :, 0:1]
    o = num / (den + 1e-16)
    t = (o[0] + o[1]) * 0.5 + bg_ref[...]
    mu = jnp.mean(t, axis=-1, keepdims=True)
    var = jnp.mean(jnp.square(t - mu), axis=-1, keepdims=True)
    hn = (t - mu) / jnp.sqrt(var + 1e-5) * lng_ref[...] + lnb_ref[...]
    hn = jax.nn.relu(hn)
    return jnp.where(valid, hn, 0.0)


def _dense_mid_body(parts_ref, den_ref, bg_ref, lng_ref, lnb_ref,
                    wl_ref, wr_ref, xlp_ref, xrp_ref):
    i = pl.program_id(0)
    rows = i * ROWB + lax.broadcasted_iota(jnp.int32, (ROWB, 1), 0)
    valid = rows < N
    h = _node_update(parts_ref[...], den_ref[...], bg_ref, lng_ref, lnb_ref,
                     valid)
    _proj_write(h, wl_ref, wr_ref, xlp_ref, xrp_ref)


def _dense_post_body(parts_ref, den_ref, bg_ref, lng_ref, lnb_ref, h_ref):
    i = pl.program_id(0)
    rows = i * ROWB + lax.broadcasted_iota(jnp.int32, (ROWB, 1), 0)
    valid = rows < N
    h_ref[...] = _node_update(parts_ref[...], den_ref[...], bg_ref, lng_ref,
                              lnb_ref, valid)


def _logit_body(gsum_ref, att_ref, a_ref):
    g = gsum_ref[...]
    att = att_ref[...]
    l0 = jnp.sum(g[0] * att[0][None, :], axis=-1)
    l1 = jnp.sum(g[1] * att[1][None, :], axis=-1)
    lg = jnp.stack([l0, l1])
    a_ref[...] = jnp.exp(jnp.minimum(lg, 50.0))


def _pool_mlp_body(h_ref, b3d_ref, w1_ref, b1_ref, w2_ref, b2_ref,
                   w3_ref, b3_ref, out_ref, acc_ref):
    i = pl.program_id(0)

    @pl.when(i == 0)
    def _():
        acc_ref[...] = jnp.zeros_like(acc_ref)

    b = b3d_ref[0, 0, :]
    onehot = (lax.broadcasted_iota(jnp.int32, (NG, ROWB), 0)
              == b[None, :]).astype(_f32)
    haug = jnp.concatenate(
        [h_ref[...], jnp.ones((ROWB, 1), _f32)], axis=-1)
    acc_ref[...] += jnp.dot(onehot, haug, preferred_element_type=_f32)

    @pl.when(i == NRB - 1)
    def _():
        acc = acc_ref[...]
        emb = acc[:, :HID] / jnp.maximum(acc[:, HID:HID + 1], 1.0)
        z = jax.nn.relu(jnp.dot(emb, w1_ref[...],
                                preferred_element_type=_f32) + b1_ref[...])
        z = jax.nn.relu(jnp.dot(z, w2_ref[...],
                                preferred_element_type=_f32) + b2_ref[...])
        z = jnp.dot(z, w3_ref[...], preferred_element_type=_f32) + b3_ref[...]
        out_ref[...] = jax.nn.sigmoid(z)


def _sc_gather_body(xl_hbm, xr_hbm, s_hbm, d_hbm, gsum_hbm,
                    sidx_a, didx_a, bufl_a, bufr_a,
                    sidx_b, didx_b, bufl_b, bufr_b,
                    seml_a, semr_a, seml_b, semr_b):
    c = lax.axis_index("c")
    s = lax.axis_index("s")
    off = c * N8

    def fire(j, sidx_v, didx_v, bufl, bufr, seml, semr):
        base = (s * CPS_G + j) * CHUNK_G
        pltpu.sync_copy(s_hbm.at[pl.ds(base, CHUNK_G)], sidx_v)
        pltpu.sync_copy(d_hbm.at[pl.ds(base, CHUNK_G)], didx_v)

        @plsc.parallel_loop(0, CHUNK_G // 16)
        def _(t):
            sl = pl.ds(t * 16, 16)
            sidx_v[sl] = sidx_v[sl] + off
            didx_v[sl] = didx_v[sl] + off

        pltpu.async_copy(xl_hbm.at[sidx_v], bufl, seml)
        pltpu.async_copy(xr_hbm.at[didx_v], bufr, semr)

    def proc(j, sidx_v, didx_v, bufl, bufr, seml, semr):
        pltpu.make_async_copy(xl_hbm.at[sidx_v], bufl, seml).wait()
        pltpu.make_async_copy(xr_hbm.at[didx_v], bufr, semr).wait()

        @plsc.parallel_loop(0, CHUNK_G, unroll=4)
        def _(k):
            for o in (0, 16):
                sl = pl.ds(o, 16)
                v = bufl[k, sl] + bufr[k, sl]
                bufl[k, sl] = jnp.maximum(v, v * 0.2)

        base = (s * CPS_G + j) * CHUNK_G
        pltpu.sync_copy(bufl, gsum_hbm.at[pl.ds(c * EP + base, CHUNK_G)])

    fire(0, sidx_a, didx_a, bufl_a, bufr_a, seml_a, semr_a)
    fire(1, sidx_b, didx_b, bufl_b, bufr_b, seml_b, semr_b)

    def pair(jj, carry):
        j0 = 2 * jj
        proc(j0, sidx_a, didx_a, bufl_a, bufr_a, seml_a, semr_a)

        @pl.when(j0 + 2 < CPS_G)
        def _():
            fire(j0 + 2, sidx_a, didx_a, bufl_a, bufr_a, seml_a, semr_a)
        proc(j0 + 1, sidx_b, didx_b, bufl_b, bufr_b, seml_b, semr_b)

        @pl.when(j0 + 3 < CPS_G)
        def _():
            fire(j0 + 3, sidx_b, didx_b, bufl_b, bufr_b, seml_b, semr_b)
        return carry
    lax.fori_loop(0, CPS_G // 2, pair, 0)


def _sc_scatter_body(xl_hbm, s_hbm, d_hbm, a_hbm, parts_hbm,
                     sidx_a, didx_a, a_a, gbuf_a,
                     sidx_b, didx_b, a_b, gbuf_b,
                     zbuf, acc, sem_a, sem_b):
    c = lax.axis_index("c")
    s = lax.axis_index("s")
    off = c * N8

    def zrow(i, carry):
        z = jnp.zeros((16,), _f32)
        zbuf[i, pl.ds(0, 16)] = z
        zbuf[i, pl.ds(16, 16)] = z
        return carry
    lax.fori_loop(0, ZROWS, zrow, 0)
    for r in range(28):
        pltpu.sync_copy(zbuf, acc.at[pl.ds(s * STRIPE + r * ZROWS, ZROWS)])
    plsc.subcore_barrier()

    def fire(j, sidx_v, didx_v, a_v, gbuf, sem):
        base = (s * CPS_S + j) * CHUNK_S
        pltpu.sync_copy(s_hbm.at[pl.ds(base, CHUNK_S)], sidx_v)
        pltpu.sync_copy(d_hbm.at[pl.ds(base, CHUNK_S)], didx_v)
        pltpu.sync_copy(a_hbm.at[pl.ds(c * EP + base, CHUNK_S)], a_v)

        @plsc.parallel_loop(0, CHUNK_S // 16)
        def _(t):
            sl = pl.ds(t * 16, 16)
            sidx_v[sl] = sidx_v[sl] + off

        pltpu.async_copy(xl_hbm.at[sidx_v], gbuf, sem)

    def proc(sidx_v, didx_v, a_v, gbuf, sem):
        pltpu.make_async_copy(xl_hbm.at[sidx_v], gbuf, sem).wait()

        @plsc.parallel_loop(0, CHUNK_S // 16, unroll=2)
        def _(t):
            av = a_v[pl.ds(t * 16, 16)]
            s0, s1 = pl.ds(0, 16), pl.ds(16, 16)
            for i in range(16):
                k = t * 16 + i
                ak = av[i]
                v0, v1 = gbuf[k, s0], gbuf[k, s1]
                gbuf[k, s0] = v0 * ak
                gbuf[k, s1] = v1 * ak

        pltpu.sync_copy(gbuf, acc.at[didx_v], add=True)

    fire(0, sidx_a, didx_a, a_a, gbuf_a, sem_a)
    fire(1, sidx_b, didx_b, a_b, gbuf_b, sem_b)

    def pair(jj, carry):
        j0 = 2 * jj
        proc(sidx_a, didx_a, a_a, gbuf_a, sem_a)

        @pl.when(j0 + 2 < CPS_S)
        def _():
            fire(j0 + 2, sidx_a, didx_a, a_a, gbuf_a, sem_a)
        proc(sidx_b, didx_b, a_b, gbuf_b, sem_b)

        @pl.when(j0 + 3 < CPS_S)
        def _():
            fire(j0 + 3, sidx_b, didx_b, a_b, gbuf_b, sem_b)
        return carry
    lax.fori_loop(0, CPS_S // 2, pair, 0)

    plsc.subcore_barrier()
    pltpu.sync_copy(acc.at[pl.ds(s * STRIPE, STRIPE)],
                    parts_hbm.at[pl.ds(c * N8 + s * STRIPE, STRIPE)])


def _sc_den_body(d_hbm, a_hbm, den_hbm, didx_v, a_v, denbuf, zbuf, acc):
    c = lax.axis_index("c")
    s = lax.axis_index("s")
    iota16 = jnp.arange(16, dtype=jnp.int32)
    zero16 = jnp.zeros((16,), jnp.int32)

    def zrow(i, carry):
        zbuf[i, pl.ds(0, 16)] = jnp.zeros((16,), _f32)
        return carry
    lax.fori_loop(0, ZROWS, zrow, 0)

    def zden(i, carry):
        denbuf[i, pl.ds(0, 16)] = jnp.zeros((16,), _f32)
        return carry
    lax.fori_loop(0, CHUNK_S, zden, 0)
    for r in range(28):
        pltpu.sync_copy(zbuf, acc.at[pl.ds(s * STRIPE + r * ZROWS, ZROWS)])
    plsc.subcore_barrier()

    def chunk_body(j, carry):
        base = (s * CPS_S + j) * CHUNK_S
        pltpu.sync_copy(d_hbm.at[pl.ds(base, CHUNK_S)], didx_v)
        pltpu.sync_copy(a_hbm.at[pl.ds(c * EP + base, CHUNK_S)], a_v)

        def grp(t, carry2):
            av = a_v[pl.ds(t * 16, 16)]
            plsc.store_scatter(denbuf, [iota16 + t * 16, zero16], av)
            return carry2
        lax.fori_loop(0, CHUNK_S // 16, grp, 0)

        pltpu.sync_copy(denbuf, acc.at[didx_v], add=True)
        return carry
    lax.fori_loop(0, CPS_S, chunk_body, 0)

    plsc.subcore_barrier()
    pltpu.sync_copy(acc.at[pl.ds(s * STRIPE, STRIPE)],
                    den_hbm.at[pl.ds(c * N8 + s * STRIPE, STRIPE)])


_SC_MESH = plsc.VectorSubcoreMesh(core_axis_name="c", subcore_axis_name="s")
_SC_PARAMS = pltpu.CompilerParams(use_tc_tiling_on_sc=False)

_sc_gather = pl.kernel(
    _sc_gather_body,
    out_type=jax.ShapeDtypeStruct((H * EP, C), _f32),
    mesh=_SC_MESH,
    compiler_params=_SC_PARAMS,
    scratch_types=[
        pltpu.VMEM((CHUNK_G,), jnp.int32),
        pltpu.VMEM((CHUNK_G,), jnp.int32),
        pltpu.VMEM((CHUNK_G, C), _f32),
        pltpu.VMEM((CHUNK_G, C), _f32),
        pltpu.VMEM((CHUNK_G,), jnp.int32),
        pltpu.VMEM((CHUNK_G,), jnp.int32),
        pltpu.VMEM((CHUNK_G, C), _f32),
        pltpu.VMEM((CHUNK_G, C), _f32),
        pltpu.SemaphoreType.DMA,
        pltpu.SemaphoreType.DMA,
        pltpu.SemaphoreType.DMA,
        pltpu.SemaphoreType.DMA,
    ],
)

_sc_scatter = pl.kernel(
    _sc_scatter_body,
    out_type=jax.ShapeDtypeStruct((H * N8, C), _f32),
    mesh=_SC_MESH,
    compiler_params=_SC_PARAMS,
    scratch_types=[
        pltpu.VMEM((CHUNK_S,), jnp.int32),
        pltpu.VMEM((CHUNK_S,), jnp.int32),
        pltpu.VMEM((CHUNK_S,), _f32),
        pltpu.VMEM((CHUNK_S, C), _f32),
        pltpu.VMEM((CHUNK_S,), jnp.int32),
        pltpu.VMEM((CHUNK_S,), jnp.int32),
        pltpu.VMEM((CHUNK_S,), _f32),
        pltpu.VMEM((CHUNK_S, C), _f32),
        pltpu.VMEM((ZROWS, C), _f32),
        pltpu.VMEM_SHARED((N8, C), _f32),
        pltpu.SemaphoreType.DMA,
        pltpu.SemaphoreType.DMA,
    ],
)

_sc_den = pl.kernel(
    _sc_den_body,
    out_type=jax.ShapeDtypeStruct((H * N8, DENW), _f32),
    mesh=_SC_MESH,
    compiler_params=pltpu.CompilerParams(
        use_tc_tiling_on_sc=False, needs_layout_passes=False),
    scratch_types=[
        pltpu.VMEM((CHUNK_S,), jnp.int32),
        pltpu.VMEM((CHUNK_S,), _f32),
        pltpu.VMEM((CHUNK_S, DENW), _f32),
        pltpu.VMEM((ZROWS, DENW), _f32),
        pltpu.VMEM_SHARED((N8, DENW), _f32),
    ],
)


def _full(shape):
    return pl.BlockSpec(shape, lambda i: tuple(0 for _ in shape))


def _dense_pre(x_pad, w_in, b_in, wl, wr):
    return pl.pallas_call(
        _dense_pre_body,
        grid=(NRB,),
        in_specs=[
            pl.BlockSpec((ROWB, F_IN), lambda i: (i, 0)),
            _full((F_IN, HID)),
            _full((1, HID)),
            _full((HID, H * C)),
            _full((HID, H * C)),
        ],
        out_specs=[
            pl.BlockSpec((H, ROWB, C), lambda i: (0, i, 0)),
            pl.BlockSpec((H, ROWB, C), lambda i: (0, i, 0)),
        ],
        out_shape=[
            jax.ShapeDtypeStruct((H, N8, C), _f32),
            jax.ShapeDtypeStruct((H, N8, C), _f32),
        ],
    )(x_pad, w_in, b_in, wl, wr)


def _dense_mid(parts, den, bg, lng, lnb, wl, wr):
    return pl.pallas_call(
        _dense_mid_body,
        grid=(NRB,),
        in_specs=[
            pl.BlockSpec((H, ROWB, C), lambda i: (0, i, 0)),
            pl.BlockSpec((H, ROWB, DENW), lambda i: (0, i, 0)),
            _full((1, C)),
            _full((1, HID)),
            _full((1, HID)),
            _full((HID, H * C)),
            _full((HID, H * C)),
        ],
        out_specs=[
            pl.BlockSpec((H, ROWB, C), lambda i: (0, i, 0)),
            pl.BlockSpec((H, ROWB, C), lambda i: (0, i, 0)),
        ],
        out_shape=[
            jax.ShapeDtypeStruct((H, N8, C), _f32),
            jax.ShapeDtypeStruct((H, N8, C), _f32),
        ],
    )(parts, den, bg, lng, lnb, wl, wr)


def _dense_post(parts, den, bg, lng, lnb):
    return pl.pallas_call(
        _dense_post_body,
        grid=(NRB,),
        in_specs=[
            pl.BlockSpec((H, ROWB, C), lambda i: (0, i, 0)),
            pl.BlockSpec((H, ROWB, DENW), lambda i: (0, i, 0)),
            _full((1, C)),
            _full((1, HID)),
            _full((1, HID)),
        ],
        out_specs=pl.BlockSpec((ROWB, HID), lambda i: (i, 0)),
        out_shape=jax.ShapeDtypeStruct((N8, HID), _f32),
    )(parts, den, bg, lng, lnb)


def _logits(gsum, att):
    return pl.pallas_call(
        _logit_body,
        grid=(NEB,),
        in_specs=[
            pl.BlockSpec((H, EB, C), lambda i: (0, i, 0)),
            _full((H, C)),
        ],
        out_specs=pl.BlockSpec((H, EB), lambda i: (0, i)),
        out_shape=jax.ShapeDtypeStruct((H, EP), _f32),
    )(gsum, att)


def _pool_mlp(h2, batch3d, w1, b1, w2, b2, w3, b3):
    return pl.pallas_call(
        _pool_mlp_body,
        grid=(NRB,),
        in_specs=[
            pl.BlockSpec((ROWB, HID), lambda i: (i, 0)),
            pl.BlockSpec((1, 1, ROWB), lambda i: (i, 0, 0)),
            _full((HID, 128)),
            _full((1, 128)),
            _full((128, 64)),
            _full((1, 64)),
            _full((64, 1)),
            _full((1, 1)),
        ],
        out_specs=pl.BlockSpec((NG, 1), lambda i: (0, 0)),
        out_shape=jax.ShapeDtypeStruct((NG, 1), _f32),
        scratch_shapes=[pltpu.VMEM((NG, HID + 1), _f32)],
    )(h2, batch3d, w1, b1, w2, b2, w3, b3)


def _gat_layer(xlp, xrp, sidx, didx, att):
    xl_flat = xlp.reshape(H * N8, C)
    xr_flat = xrp.reshape(H * N8, C)
    gsum = _sc_gather(xl_flat, xr_flat, sidx, didx)
    a = _logits(gsum.reshape(H, EP, C), att).reshape(H * EP)
    parts = _sc_scatter(xl_flat, sidx, didx, a)
    den = _sc_den(didx, a)
    return parts.reshape(H, N8, C), den.reshape(H, N8, DENW)


def kernel(x, edge_index, batch, W_in, b_in, Wl0, Wr0, att0, bg0, lng0, lnb0,
           Wl1, Wr1, att1, bg1, lng1, lnb1, W1, b1, W2, b2, W3, b3):
    loop = jnp.arange(N, dtype=jnp.int32)
    padv = jnp.full((EP - E - N,), N, jnp.int32)
    sidx = jnp.concatenate([edge_index[0], loop, padv])
    didx = jnp.concatenate([edge_index[1], loop, padv])
    x_pad = jnp.pad(x, ((0, N8 - N), (0, 0)))
    batch3d = jnp.concatenate(
        [batch, jnp.full((N8 - N,), NG, jnp.int32)]).reshape(NRB, 1, ROWB)

    xlp0, xrp0 = _dense_pre(x_pad, W_in, b_in.reshape(1, HID), Wl0, Wr0)
    parts0, den0 = _gat_layer(xlp0, xrp0, sidx, didx, att0)

    xlp1, xrp1 = _dense_mid(
        parts0, den0, bg0.reshape(1, C), lng0.reshape(1, HID),
        lnb0.reshape(1, HID), Wl1, Wr1)
    parts1, den1 = _gat_layer(xlp1, xrp1, sidx, didx, att1)

    h2 = _dense_post(
        parts1, den1, bg1.reshape(1, C), lng1.reshape(1, HID),
        lnb1.reshape(1, HID))

    return _pool_mlp(h2, batch3d, W1, b1.reshape(1, 128), W2,
                     b2.reshape(1, 64), W3, b3.reshape(1, 1))


# speedup vs baseline: 44.7758x; 1.2142x over previous
import functools

import jax
import jax.numpy as jnp
from jax import lax
from jax.experimental import pallas as pl
from jax.experimental.pallas import tpu as pltpu
from jax.experimental.pallas import tpu_sc as plsc

N = 50000
E = 800000
F_IN = 2
HID = 32
H = 2
C = 32
NG = 64

N8 = 50176
ROWB = 1024
NRB = N8 // ROWB
STRIPE = N8 // 16
ZROWS = STRIPE // 28
DENW = 16

CHUNK = 256
NSUB = 16
CPS = -(-(E + N) // (NSUB * CHUNK))
EP = NSUB * CPS * CHUNK
CHUNK_G = 512
CPS_G = EP // (NSUB * CHUNK_G)
AUGW = 36
EB = 16384
NEB = EP // EB
CHUNK_S = 128
CPS_S = EP // (NSUB * CHUNK_S)

_f32 = jnp.float32


def _proj_write(h, wl_ref, wr_ref, xlp_ref, xrp_ref):
    xl = jnp.dot(h, wl_ref[...], preferred_element_type=_f32)
    xr = jnp.dot(h, wr_ref[...], preferred_element_type=_f32)
    xlp_ref[...] = xl.reshape(ROWB, H, C).transpose(1, 0, 2)
    xrp_ref[...] = xr.reshape(ROWB, H, C).transpose(1, 0, 2)


def _dense_pre_body(x_ref, win_ref, bin_ref, wl_ref, wr_ref,
                    xlp_ref, xrp_ref):
    i = pl.program_id(0)
    rows = i * ROWB + lax.broadcasted_iota(jnp.int32, (ROWB, 1), 0)
    valid = rows < N
    h = jax.nn.relu(jnp.dot(x_ref[...], win_ref[...],
                            preferred_element_type=_f32) + bin_ref[...])
    h = jnp.where(valid, h, 0.0)
    _proj_write(h, wl_ref, wr_ref, xlp_ref, xrp_ref)


def _node_update(parts, den3, bg_ref, lng_ref, lnb_ref, valid):
    num = parts
    den = den3[:, :, 0:1]
    o = num / (den + 1e-16)
    t = (o[0] + o[1]) * 0.5 + bg_ref[...]
    mu = jnp.mean(t, axis=-1, keepdims=True)
    var = jnp.mean(jnp.square(t - mu), axis=-1, keepdims=True)
    hn = (t - mu) / jnp.sqrt(var + 1e-5) * lng_ref[...] + lnb_ref[...]
    hn = jax.nn.relu(hn)
    return jnp.where(valid, hn, 0.0)


def _dense_mid_body(parts_ref, den_ref, bg_ref, lng_ref, lnb_ref,
                    wl_ref, wr_ref, xlp_ref, xrp_ref):
    i = pl.program_id(0)
    rows = i * ROWB + lax.broadcasted_iota(jnp.int32, (ROWB, 1), 0)
    valid = rows < N
    h = _node_update(parts_ref[...], den_ref[...], bg_ref, lng_ref, lnb_ref,
                     valid)
    _proj_write(h, wl_ref, wr_ref, xlp_ref, xrp_ref)


def _dense_post_body(parts_ref, den_ref, bg_ref, lng_ref, lnb_ref, h_ref):
    i = pl.program_id(0)
    rows = i * ROWB + lax.broadcasted_iota(jnp.int32, (ROWB, 1), 0)
    valid = rows < N
    h_ref[...] = _node_update(parts_ref[...], den_ref[...], bg_ref, lng_ref,
                              lnb_ref, valid)


def _logit_body(gsum_ref, att_ref, a_ref):
    g = gsum_ref[...]
    att = att_ref[...]
    l0 = jnp.sum(g[0] * att[0][None, :], axis=-1)
    l1 = jnp.sum(g[1] * att[1][None, :], axis=-1)
    lg = jnp.stack([l0, l1])
    a_ref[...] = jnp.exp(jnp.minimum(lg, 50.0))


def _pool_mlp_body(h_ref, b3d_ref, w1_ref, b1_ref, w2_ref, b2_ref,
                   w3_ref, b3_ref, out_ref, acc_ref):
    i = pl.program_id(0)

    @pl.when(i == 0)
    def _():
        acc_ref[...] = jnp.zeros_like(acc_ref)

    b = b3d_ref[0, 0, :]
    onehot = (lax.broadcasted_iota(jnp.int32, (NG, ROWB), 0)
              == b[None, :]).astype(_f32)
    haug = jnp.concatenate(
        [h_ref[...], jnp.ones((ROWB, 1), _f32)], axis=-1)
    acc_ref[...] += jnp.dot(onehot, haug, preferred_element_type=_f32)

    @pl.when(i == NRB - 1)
    def _():
        acc = acc_ref[...]
        emb = acc[:, :HID] / jnp.maximum(acc[:, HID:HID + 1], 1.0)
        z = jax.nn.relu(jnp.dot(emb, w1_ref[...],
                                preferred_element_type=_f32) + b1_ref[...])
        z = jax.nn.relu(jnp.dot(z, w2_ref[...],
                                preferred_element_type=_f32) + b2_ref[...])
        z = jnp.dot(z, w3_ref[...], preferred_element_type=_f32) + b3_ref[...]
        out_ref[...] = jax.nn.sigmoid(z)


def _sc_gather_body(xl_hbm, xr_hbm, s_hbm, d_hbm, gsum_hbm,
                    sidx_a, didx_a, bufl_a, bufr_a,
                    sidx_b, didx_b, bufl_b, bufr_b,
                    seml_a, semr_a, seml_b, semr_b):
    c = lax.axis_index("c")
    s = lax.axis_index("s")
    off = c * N8

    def fire(j, sidx_v, didx_v, bufl, bufr, seml, semr):
        base = (s * CPS_G + j) * CHUNK_G
        pltpu.sync_copy(s_hbm.at[pl.ds(base, CHUNK_G)], sidx_v)
        pltpu.sync_copy(d_hbm.at[pl.ds(base, CHUNK_G)], didx_v)

        @plsc.parallel_loop(0, CHUNK_G // 16)
        def _(t):
            sl = pl.ds(t * 16, 16)
            sidx_v[sl] = sidx_v[sl] + off
            didx_v[sl] = didx_v[sl] + off

        pltpu.async_copy(xl_hbm.at[sidx_v], bufl, seml)
        pltpu.async_copy(xr_hbm.at[didx_v], bufr, semr)

    def proc(j, sidx_v, didx_v, bufl, bufr, seml, semr):
        pltpu.make_async_copy(xl_hbm.at[sidx_v], bufl, seml).wait()
        pltpu.make_async_copy(xr_hbm.at[didx_v], bufr, semr).wait()

        @plsc.parallel_loop(0, CHUNK_G, unroll=4)
        def _(k):
            for o in (0, 16):
                sl = pl.ds(o, 16)
                v = bufl[k, sl] + bufr[k, sl]
                bufl[k, sl] = jnp.maximum(v, v * 0.2)

        base = (s * CPS_G + j) * CHUNK_G
        pltpu.sync_copy(bufl, gsum_hbm.at[pl.ds(c * EP + base, CHUNK_G)])

    fire(0, sidx_a, didx_a, bufl_a, bufr_a, seml_a, semr_a)
    fire(1, sidx_b, didx_b, bufl_b, bufr_b, seml_b, semr_b)

    def pair(jj, carry):
        j0 = 2 * jj
        proc(j0, sidx_a, didx_a, bufl_a, bufr_a, seml_a, semr_a)

        @pl.when(j0 + 2 < CPS_G)
        def _():
            fire(j0 + 2, sidx_a, didx_a, bufl_a, bufr_a, seml_a, semr_a)
        proc(j0 + 1, sidx_b, didx_b, bufl_b, bufr_b, seml_b, semr_b)

        @pl.when(j0 + 3 < CPS_G)
        def _():
            fire(j0 + 3, sidx_b, didx_b, bufl_b, bufr_b, seml_b, semr_b)
        return carry
    lax.fori_loop(0, CPS_G // 2, pair, 0)


def _sc_scatter_body(xl_hbm, s_hbm, d_hbm, a_hbm, parts_hbm,
                     sidx_a, didx_a, a_a, gbuf_a,
                     sidx_b, didx_b, a_b, gbuf_b,
                     zbuf, acc, sem_a, sem_b):
    c = lax.axis_index("c")
    s = lax.axis_index("s")
    off = c * N8

    def zrow(i, carry):
        z = jnp.zeros((16,), _f32)
        zbuf[i, pl.ds(0, 16)] = z
        zbuf[i, pl.ds(16, 16)] = z
        return carry
    lax.fori_loop(0, ZROWS, zrow, 0)
    for r in range(28):
        pltpu.sync_copy(zbuf, acc.at[pl.ds(s * STRIPE + r * ZROWS, ZROWS)])
    plsc.subcore_barrier()

    def fire(j, sidx_v, didx_v, a_v, gbuf, sem):
        base = (s * CPS_S + j) * CHUNK_S
        pltpu.sync_copy(s_hbm.at[pl.ds(base, CHUNK_S)], sidx_v)
        pltpu.sync_copy(d_hbm.at[pl.ds(base, CHUNK_S)], didx_v)
        pltpu.sync_copy(a_hbm.at[pl.ds(c * EP + base, CHUNK_S)], a_v)

        @plsc.parallel_loop(0, CHUNK_S // 16)
        def _(t):
            sl = pl.ds(t * 16, 16)
            sidx_v[sl] = sidx_v[sl] + off

        pltpu.async_copy(xl_hbm.at[sidx_v], gbuf, sem)

    def proc(sidx_v, didx_v, a_v, gbuf, sem):
        pltpu.make_async_copy(xl_hbm.at[sidx_v], gbuf, sem).wait()

        @plsc.parallel_loop(0, CHUNK_S // 16, unroll=2)
        def _(t):
            av = a_v[pl.ds(t * 16, 16)]
            s0, s1 = pl.ds(0, 16), pl.ds(16, 16)
            for i in range(16):
                k = t * 16 + i
                ak = av[i]
                v0, v1 = gbuf[k, s0], gbuf[k, s1]
                gbuf[k, s0] = v0 * ak
                gbuf[k, s1] = v1 * ak

        pltpu.sync_copy(gbuf, acc.at[didx_v], add=True)

    fire(0, sidx_a, didx_a, a_a, gbuf_a, sem_a)
    fire(1, sidx_b, didx_b, a_b, gbuf_b, sem_b)

    def pair(jj, carry):
        j0 = 2 * jj
        proc(sidx_a, didx_a, a_a, gbuf_a, sem_a)

        @pl.when(j0 + 2 < CPS_S)
        def _():
            fire(j0 + 2, sidx_a, didx_a, a_a, gbuf_a, sem_a)
        proc(sidx_b, didx_b, a_b, gbuf_b, sem_b)

        @pl.when(j0 + 3 < CPS_S)
        def _():
            fire(j0 + 3, sidx_b, didx_b, a_b, gbuf_b, sem_b)
        return carry
    lax.fori_loop(0, CPS_S // 2, pair, 0)

    plsc.subcore_barrier()
    pltpu.sync_copy(acc.at[pl.ds(s * STRIPE, STRIPE)],
                    parts_hbm.at[pl.ds(c * N8 + s * STRIPE, STRIPE)])


def _sc_den_body(d_hbm, a_hbm, den_hbm,
                 didx_a, a_a, dbuf_a, didx_b, a_b, dbuf_b,
                 zbuf, acc, sem_a, sem_b):
    c = lax.axis_index("c")
    s = lax.axis_index("s")
    iota16 = jnp.arange(16, dtype=jnp.int32)
    zero16 = jnp.zeros((16,), jnp.int32)

    def zrow(i, carry):
        zbuf[i, pl.ds(0, 16)] = jnp.zeros((16,), _f32)
        return carry
    lax.fori_loop(0, ZROWS, zrow, 0)

    for dbuf in (dbuf_a, dbuf_b):
        def zden(i, carry, _dbuf=dbuf):
            _dbuf[i, pl.ds(0, 16)] = jnp.zeros((16,), _f32)
            return carry
        lax.fori_loop(0, CHUNK_S, zden, 0)
    for r in range(28):
        pltpu.sync_copy(zbuf, acc.at[pl.ds(s * STRIPE + r * ZROWS, ZROWS)])
    plsc.subcore_barrier()

    def fire(j, didx_v, a_v, sem):
        base = (s * CPS_S + j) * CHUNK_S
        pltpu.async_copy(d_hbm.at[pl.ds(base, CHUNK_S)], didx_v, sem)
        pltpu.async_copy(a_hbm.at[pl.ds(c * EP + base, CHUNK_S)], a_v, sem)

    def proc(j, didx_v, a_v, dbuf, sem):
        base = (s * CPS_S + j) * CHUNK_S
        pltpu.make_async_copy(d_hbm.at[pl.ds(base, CHUNK_S)], didx_v,
                              sem).wait()
        pltpu.make_async_copy(a_hbm.at[pl.ds(c * EP + base, CHUNK_S)], a_v,
                              sem).wait()

        @plsc.parallel_loop(0, CHUNK_S // 16, unroll=2)
        def _(t):
            av = a_v[pl.ds(t * 16, 16)]
            plsc.store_scatter(dbuf, [iota16 + t * 16, zero16], av)

        pltpu.sync_copy(dbuf, acc.at[didx_v], add=True)

    fire(0, didx_a, a_a, sem_a)
    fire(1, didx_b, a_b, sem_b)

    def pair(jj, carry):
        j0 = 2 * jj
        proc(j0, didx_a, a_a, dbuf_a, sem_a)

        @pl.when(j0 + 2 < CPS_S)
        def _():
            fire(j0 + 2, didx_a, a_a, sem_a)
        proc(j0 + 1, didx_b, a_b, dbuf_b, sem_b)

        @pl.when(j0 + 3 < CPS_S)
        def _():
            fire(j0 + 3, didx_b, a_b, sem_b)
        return carry
    lax.fori_loop(0, CPS_S // 2, pair, 0)

    plsc.subcore_barrier()
    pltpu.sync_copy(acc.at[pl.ds(s * STRIPE, STRIPE)],
                    den_hbm.at[pl.ds(c * N8 + s * STRIPE, STRIPE)])


_SC_MESH = plsc.VectorSubcoreMesh(core_axis_name="c", subcore_axis_name="s")
_SC_PARAMS = pltpu.CompilerParams(use_tc_tiling_on_sc=False)

_sc_gather = pl.kernel(
    _sc_gather_body,
    out_type=jax.ShapeDtypeStruct((H * EP, C), _f32),
    mesh=_SC_MESH,
    compiler_params=_SC_PARAMS,
    scratch_types=[
        pltpu.VMEM((CHUNK_G,), jnp.int32),
        pltpu.VMEM((CHUNK_G,), jnp.int32),
        pltpu.VMEM((CHUNK_G, C), _f32),
        pltpu.VMEM((CHUNK_G, C), _f32),
        pltpu.VMEM((CHUNK_G,), jnp.int32),
        pltpu.VMEM((CHUNK_G,), jnp.int32),
        pltpu.VMEM((CHUNK_G, C), _f32),
        pltpu.VMEM((CHUNK_G, C), _f32),
        pltpu.SemaphoreType.DMA,
        pltpu.SemaphoreType.DMA,
        pltpu.SemaphoreType.DMA,
        pltpu.SemaphoreType.DMA,
    ],
)

_sc_scatter = pl.kernel(
    _sc_scatter_body,
    out_type=jax.ShapeDtypeStruct((H * N8, C), _f32),
    mesh=_SC_MESH,
    compiler_params=_SC_PARAMS,
    scratch_types=[
        pltpu.VMEM((CHUNK_S,), jnp.int32),
        pltpu.VMEM((CHUNK_S,), jnp.int32),
        pltpu.VMEM((CHUNK_S,), _f32),
        pltpu.VMEM((CHUNK_S, C), _f32),
        pltpu.VMEM((CHUNK_S,), jnp.int32),
        pltpu.VMEM((CHUNK_S,), jnp.int32),
        pltpu.VMEM((CHUNK_S,), _f32),
        pltpu.VMEM((CHUNK_S, C), _f32),
        pltpu.VMEM((ZROWS, C), _f32),
        pltpu.VMEM_SHARED((N8, C), _f32),
        pltpu.SemaphoreType.DMA,
        pltpu.SemaphoreType.DMA,
    ],
)

_sc_den = pl.kernel(
    _sc_den_body,
    out_type=jax.ShapeDtypeStruct((H * N8, DENW), _f32),
    mesh=_SC_MESH,
    compiler_params=pltpu.CompilerParams(
        use_tc_tiling_on_sc=False, needs_layout_passes=False),
    scratch_types=[
        pltpu.VMEM((CHUNK_S,), jnp.int32),
        pltpu.VMEM((CHUNK_S,), _f32),
        pltpu.VMEM((CHUNK_S, DENW), _f32),
        pltpu.VMEM((CHUNK_S,), jnp.int32),
        pltpu.VMEM((CHUNK_S,), _f32),
        pltpu.VMEM((CHUNK_S, DENW), _f32),
        pltpu.VMEM((ZROWS, DENW), _f32),
        pltpu.VMEM_SHARED((N8, DENW), _f32),
        pltpu.SemaphoreType.DMA,
        pltpu.SemaphoreType.DMA,
    ],
)


def _full(shape):
    return pl.BlockSpec(shape, lambda i: tuple(0 for _ in shape))


def _dense_pre(x_pad, w_in, b_in, wl, wr):
    return pl.pallas_call(
        _dense_pre_body,
        grid=(NRB,),
        in_specs=[
            pl.BlockSpec((ROWB, F_IN), lambda i: (i, 0)),
            _full((F_IN, HID)),
            _full((1, HID)),
            _full((HID, H * C)),
            _full((HID, H * C)),
        ],
        out_specs=[
            pl.BlockSpec((H, ROWB, C), lambda i: (0, i, 0)),
            pl.BlockSpec((H, ROWB, C), lambda i: (0, i, 0)),
        ],
        out_shape=[
            jax.ShapeDtypeStruct((H, N8, C), _f32),
            jax.ShapeDtypeStruct((H, N8, C), _f32),
        ],
    )(x_pad, w_in, b_in, wl, wr)


def _dense_mid(parts, den, bg, lng, lnb, wl, wr):
    return pl.pallas_call(
        _dense_mid_body,
        grid=(NRB,),
        in_specs=[
            pl.BlockSpec((H, ROWB, C), lambda i: (0, i, 0)),
            pl.BlockSpec((H, ROWB, DENW), lambda i: (0, i, 0)),
            _full((1, C)),
            _full((1, HID)),
            _full((1, HID)),
            _full((HID, H * C)),
            _full((HID, H * C)),
        ],
        out_specs=[
            pl.BlockSpec((H, ROWB, C), lambda i: (0, i, 0)),
            pl.BlockSpec((H, ROWB, C), lambda i: (0, i, 0)),
        ],
        out_shape=[
            jax.ShapeDtypeStruct((H, N8, C), _f32),
            jax.ShapeDtypeStruct((H, N8, C), _f32),
        ],
    )(parts, den, bg, lng, lnb, wl, wr)


def _dense_post(parts, den, bg, lng, lnb):
    return pl.pallas_call(
        _dense_post_body,
        grid=(NRB,),
        in_specs=[
            pl.BlockSpec((H, ROWB, C), lambda i: (0, i, 0)),
            pl.BlockSpec((H, ROWB, DENW), lambda i: (0, i, 0)),
            _full((1, C)),
            _full((1, HID)),
            _full((1, HID)),
        ],
        out_specs=pl.BlockSpec((ROWB, HID), lambda i: (i, 0)),
        out_shape=jax.ShapeDtypeStruct((N8, HID), _f32),
    )(parts, den, bg, lng, lnb)


def _logits(gsum, att):
    return pl.pallas_call(
        _logit_body,
        grid=(NEB,),
        in_specs=[
            pl.BlockSpec((H, EB, C), lambda i: (0, i, 0)),
            _full((H, C)),
        ],
        out_specs=pl.BlockSpec((H, EB), lambda i: (0, i)),
        out_shape=jax.ShapeDtypeStruct((H, EP), _f32),
    )(gsum, att)


def _pool_mlp(h2, batch3d, w1, b1, w2, b2, w3, b3):
    return pl.pallas_call(
        _pool_mlp_body,
        grid=(NRB,),
        in_specs=[
            pl.BlockSpec((ROWB, HID), lambda i: (i, 0)),
            pl.BlockSpec((1, 1, ROWB), lambda i: (i, 0, 0)),
            _full((HID, 128)),
            _full((1, 128)),
            _full((128, 64)),
            _full((1, 64)),
            _full((64, 1)),
            _full((1, 1)),
        ],
        out_specs=pl.BlockSpec((NG, 1), lambda i: (0, 0)),
        out_shape=jax.ShapeDtypeStruct((NG, 1), _f32),
        scratch_shapes=[pltpu.VMEM((NG, HID + 1), _f32)],
    )(h2, batch3d, w1, b1, w2, b2, w3, b3)


def _gat_layer(xlp, xrp, sidx, didx, att):
    xl_flat = xlp.reshape(H * N8, C)
    xr_flat = xrp.reshape(H * N8, C)
    gsum = _sc_gather(xl_flat, xr_flat, sidx, didx)
    a = _logits(gsum.reshape(H, EP, C), att).reshape(H * EP)
    parts = _sc_scatter(xl_flat, sidx, didx, a)
    den = _sc_den(didx, a)
    return parts.reshape(H, N8, C), den.reshape(H, N8, DENW)


def kernel(x, edge_index, batch, W_in, b_in, Wl0, Wr0, att0, bg0, lng0, lnb0,
           Wl1, Wr1, att1, bg1, lng1, lnb1, W1, b1, W2, b2, W3, b3):
    loop = jnp.arange(N, dtype=jnp.int32)
    padv = jnp.full((EP - E - N,), N, jnp.int32)
    sidx = jnp.concatenate([edge_index[0], loop, padv])
    didx = jnp.concatenate([edge_index[1], loop, padv])
    x_pad = jnp.pad(x, ((0, N8 - N), (0, 0)))
    batch3d = jnp.concatenate(
        [batch, jnp.full((N8 - N,), NG, jnp.int32)]).reshape(NRB, 1, ROWB)

    xlp0, xrp0 = _dense_pre(x_pad, W_in, b_in.reshape(1, HID), Wl0, Wr0)
    parts0, den0 = _gat_layer(xlp0, xrp0, sidx, didx, att0)

    xlp1, xrp1 = _dense_mid(
        parts0, den0, bg0.reshape(1, C), lng0.reshape(1, HID),
        lnb0.reshape(1, HID), Wl1, Wr1)
    parts1, den1 = _gat_layer(xlp1, xrp1, sidx, didx, att1)

    h2 = _dense_post(
        parts1, den1, bg1.reshape(1, C), lng1.reshape(1, HID),
        lnb1.reshape(1, HID))

    return _pool_mlp(h2, batch3d, W1, b1.reshape(1, 128), W2,
                     b2.reshape(1, 64), W3, b3.reshape(1, 1))
